# baseline (speedup 1.0000x reference)
import functools

import jax
import jax.numpy as jnp
from jax import lax
from jax.experimental import pallas as pl
from jax.experimental.pallas import tpu as pltpu
from jax.experimental.pallas import tpu_sc as plsc

N = 10000
D = 128
E = 320000
NC, NS, L = 2, 16, 16
NW = NC * NS
EPW = E // NW
CK = 80
NCHUNK = EPW // CK
RPT = 640
f32 = jnp.float32
i32 = jnp.int32


def _front_body(x_ref, w_ref, att_ref, h_ref, ap_ref, mx_ref):
    i = pl.program_id(0)
    h = jnp.dot(x_ref[...], w_ref[...], preferred_element_type=f32)
    ap = jnp.dot(h, att_ref[...], preferred_element_type=f32)
    h_ref[...] = jnp.concatenate([h, ap], axis=1)
    ap_ref[...] = ap
    cur = jnp.max(ap, axis=0, keepdims=True)

    @pl.when(i == 0)
    def _():
        mx_ref[...] = cur

    @pl.when(i != 0)
    def _():
        mx_ref[...] = jnp.maximum(mx_ref[...], cur)


def _front(x, W, att2):
    BN = 1000
    return pl.pallas_call(
        _front_body,
        grid=(N // BN,),
        in_specs=[pl.BlockSpec((BN, D), lambda i: (i, 0)),
                  pl.BlockSpec((D, D), lambda i: (0, 0)),
                  pl.BlockSpec((D, 16), lambda i: (0, 0))],
        out_specs=[pl.BlockSpec((BN, D + 16), lambda i: (i, 0)),
                   pl.BlockSpec((BN, 16), lambda i: (i, 0)),
                   pl.BlockSpec((1, 16), lambda i: (0, 0))],
        out_shape=[jax.ShapeDtypeStruct((N, D + 16), f32),
                   jax.ShapeDtypeStruct((N, 16), f32),
                   jax.ShapeDtypeStruct((1, 16), f32)],
    )(x, W, att2)


_sc_mesh = plsc.VectorSubcoreMesh(core_axis_name="c", subcore_axis_name="s",
                                  num_cores=NC, num_subcores=NS)


@functools.partial(
    pl.kernel,
    out_type=[jax.ShapeDtypeStruct((NC, N, D + 16), f32)],
    mesh=_sc_mesh,
    compiler_params=pltpu.CompilerParams(needs_layout_passes=False,
                                         use_tc_tiling_on_sc=False),
    scratch_types=[
        pltpu.VMEM((16,), f32),
        pltpu.VMEM((1, 2 * CK), i32),
        pltpu.VMEM((1, 2 * CK), i32),
        pltpu.VMEM((CK, 16), f32),
        pltpu.VMEM((CK, 16), f32),
        pltpu.VMEM((CK, D + 16), f32),
        pltpu.VMEM((CK, D + 16), f32),
        pltpu.VMEM((1, CK), i32),
        pltpu.VMEM((1, CK), i32),
        pltpu.SemaphoreType.DMA,
        pltpu.SemaphoreType.DMA,
        pltpu.SemaphoreType.DMA,
        pltpu.SemaphoreType.DMA,
        pltpu.SemaphoreType.DMA,
        pltpu.SemaphoreType.DMA,
        pltpu.VMEM_SHARED((N, D + 16), f32),
    ],
)
def _sc_edge(ap_h, c_h, ei_h, h_h,
             out_h,
             c_t, sd0, sd1, adg0, adg1, rows0, rows1,
             scx0, scx1,
             isem0, isem1, gsem0, gsem1, ssem0, ssem1,
             out_acc):
    cid = lax.axis_index("c")
    sid = lax.axis_index("s")
    wid = cid * NS + sid
    zero16 = jnp.zeros((L,), f32)

    for k in range(CK):
        for q in range((D + 16) // L):
            rows0[k, pl.ds(q * L, L)] = zero16
    nbase = sid * RPT
    for t in range(RPT // CK):
        rb = nbase + t * CK

        @pl.when(rb < N)
        def _():
            pltpu.sync_copy(rows0, out_acc.at[pl.ds(rb, CK)])

    pltpu.sync_copy(c_h, c_t)
    c_v = c_t[...]
    iot = lax.iota(i32, L)
    zeros_i = jnp.zeros((L,), i32)
    ones_i = jnp.ones((L,), i32)

    plsc.subcore_barrier()

    gdims = lax.GatherDimensionNumbers(
        offset_dims=(), collapsed_slice_dims=(0,), start_index_map=(0,))

    def _splat(v, k):
        idx = jnp.full((L, 1), k, i32)
        return lax.gather(v, idx, gdims, (1,),
                          mode=lax.GatherScatterMode.PROMISE_IN_BOUNDS)

    bufs = ((sd0, adg0, rows0, scx0, isem0, gsem0, ssem0),
            (sd1, adg1, rows1, scx1, isem1, gsem1, ssem1))
    col128 = jnp.full((L,), D, i32)

    def issue_gathers(b):
        sd, adg, rows, gsem = b[0], b[1], b[2], b[5]
        pltpu.async_copy(h_h.at[sd.at[0, pl.ds(0, CK)]], rows, gsem)
        pltpu.async_copy(ap_h.at[sd.at[0, pl.ds(CK, CK)]], adg, gsem)

    def wait_gathers(b):
        sd, adg, rows, gsem = b[0], b[1], b[2], b[5]
        pltpu.make_async_copy(h_h.at[sd.at[0, pl.ds(0, CK)]], rows, gsem).wait()
        pltpu.make_async_copy(ap_h.at[sd.at[0, pl.ds(CK, CK)]], adg, gsem).wait()

    def issue_scatters(b):
        rows, scx, ssem = b[2], b[3], b[6]
        pltpu.async_copy(rows, out_acc.at[scx.at[0]], ssem, add=True)

    def wait_scatters(b):
        rows, scx, ssem = b[2], b[3], b[6]
        pltpu.make_async_copy(rows, out_acc.at[scx.at[0]], ssem).wait()

    def process(b):
        sd, adg, rows, scx = b[0], b[1], b[2], b[3]
        wvs = []
        for g in range(CK // L):
            kvec = g * L + iot
            scx[0, pl.ds(g * L, L)] = sd[0, pl.ds(CK + g * L, L)]
            e = (plsc.load_gather(rows, [kvec, col128])
                 + plsc.load_gather(adg, [kvec, ones_i]))
            e = jnp.where(e >= 0.0, e, 0.2 * e) - c_v
            wvs.append(jnp.exp(e))
        for g in range(CK // L):
            for k in range(L):
                r = g * L + k
                wk = _splat(wvs[g], k)
                for q in range(D // L):
                    rows[r, pl.ds(q * L, L)] = rows[r, pl.ds(q * L, L)] * wk
                rows[r, pl.ds(D, L)] = wk
        issue_scatters(b)

    def step(j, X, Y, wait_scatter):
        if wait_scatter:
            wait_scatters(Y)
        pltpu.make_async_copy(ei_h.at[wid, j + 1], Y[0], Y[4]).wait()
        issue_gathers(Y)
        wait_gathers(X)
        process(X)
        jp2 = jnp.minimum(j + 2, NCHUNK - 1)
        pltpu.async_copy(ei_h.at[wid, jp2], X[0], X[4])

    pltpu.sync_copy(ei_h.at[wid, 0], sd0)
    issue_gathers(bufs[0])
    pltpu.async_copy(ei_h.at[wid, 1], sd1, isem1)
    step(0, bufs[0], bufs[1], wait_scatter=False)
    step(1, bufs[1], bufs[0], wait_scatter=True)

    def pair(p, carry):
        j = 2 * p
        step(j, bufs[0], bufs[1], wait_scatter=True)
        step(j + 1, bufs[1], bufs[0], wait_scatter=True)
        return carry

    lax.fori_loop(1, (NCHUNK - 1) // 2, pair, 0)

    wait_scatters(bufs[1])
    wait_gathers(bufs[0])
    process(bufs[0])
    wait_scatters(bufs[0])
    pltpu.make_async_copy(ei_h.at[wid, NCHUNK - 1], sd1, isem1).wait()

    plsc.subcore_barrier()

    for t in range(RPT // CK):
        rb = nbase + t * CK

        @pl.when(rb < N)
        def _():
            pltpu.sync_copy(out_acc.at[pl.ds(rb, CK)], rows0)
            pltpu.sync_copy(rows0, out_h.at[cid, pl.ds(rb, CK)])


def _mid_body(p_ref, b_ref, w_ref, att_ref, h_ref, ap_ref, mx_ref):
    i = pl.program_id(0)
    s = p_ref[0, :, 0:D] + p_ref[1, :, 0:D]
    den = p_ref[0, :, D:D + 1] + p_ref[1, :, D:D + 1]
    y = s / (den + 1e-16) + b_ref[...]
    y = jnp.where(y > 0.0, y, jnp.exp(jnp.minimum(y, 0.0)) - 1.0)
    h = jnp.dot(y, w_ref[...], preferred_element_type=f32)
    ap = jnp.dot(h, att_ref[...], preferred_element_type=f32)
    h_ref[...] = jnp.concatenate([h, ap], axis=1)
    ap_ref[...] = ap
    cur = jnp.max(ap, axis=0, keepdims=True)

    @pl.when(i == 0)
    def _():
        mx_ref[...] = cur

    @pl.when(i != 0)
    def _():
        mx_ref[...] = jnp.maximum(mx_ref[...], cur)


def _mid(p, b, W, att2):
    BN = 1000
    return pl.pallas_call(
        _mid_body,
        grid=(N // BN,),
        in_specs=[pl.BlockSpec((2, BN, D + 16), lambda i: (0, i, 0)),
                  pl.BlockSpec((1, D), lambda i: (0, 0)),
                  pl.BlockSpec((D, D), lambda i: (0, 0)),
                  pl.BlockSpec((D, 16), lambda i: (0, 0))],
        out_specs=[pl.BlockSpec((BN, D + 16), lambda i: (i, 0)),
                   pl.BlockSpec((BN, 16), lambda i: (i, 0)),
                   pl.BlockSpec((1, 16), lambda i: (0, 0))],
        out_shape=[jax.ShapeDtypeStruct((N, D + 16), f32),
                   jax.ShapeDtypeStruct((N, 16), f32),
                   jax.ShapeDtypeStruct((1, 16), f32)],
    )(p, b, W, att2)


def _combine_body(p_ref, b_ref, o_ref):
    s = p_ref[0, :, 0:D] + p_ref[1, :, 0:D]
    den = p_ref[0, :, D:D + 1] + p_ref[1, :, D:D + 1]
    o_ref[...] = s / (den + 1e-16) + b_ref[...]


def _combine(p, b):
    BN = 1000
    return pl.pallas_call(
        _combine_body,
        grid=(N // BN,),
        in_specs=[pl.BlockSpec((2, BN, D + 16), lambda i: (0, i, 0)),
                  pl.BlockSpec((1, D), lambda i: (0, 0))],
        out_specs=pl.BlockSpec((BN, D), lambda i: (i, 0)),
        out_shape=jax.ShapeDtypeStruct((N, D), f32),
    )(p, b)


def _shift(mx):
    c = mx[0, 0] + mx[0, 1]
    c = jnp.where(c >= 0.0, c, 0.2 * c)
    return jnp.broadcast_to(c, (16,))


def kernel(x, edge_index, W1, att_src1, att_dst1, b1, W2, att_src2, att_dst2, b2):
    src = edge_index[0].astype(i32).reshape(NW, NCHUNK, 1, CK)
    dst = edge_index[1].astype(i32).reshape(NW, NCHUNK, 1, CK)
    ei = jnp.concatenate([src, dst], axis=3)
    att21 = jnp.zeros((D, 16), f32).at[:, 0].set(att_src1).at[:, 1].set(att_dst1)
    att22 = jnp.zeros((D, 16), f32).at[:, 0].set(att_src2).at[:, 1].set(att_dst2)

    h1, ap1, mx1 = _front(x, W1, att21)
    outp1, = _sc_edge(ap1, _shift(mx1), ei, h1)
    h2, ap2, mx2 = _mid(outp1, b1.reshape(1, D), W2, att22)
    outp2, = _sc_edge(ap2, _shift(mx2), ei, h2)
    return _combine(outp2, b2.reshape(1, D))


# speedup vs baseline: 1.0634x; 1.0634x over previous
import functools

import jax
import jax.numpy as jnp
from jax import lax
from jax.experimental import pallas as pl
from jax.experimental.pallas import tpu as pltpu
from jax.experimental.pallas import tpu_sc as plsc

N = 10000
D = 128
E = 320000
NC, NS, L = 2, 16, 16
NW = NC * NS
EPW = E // NW
CK = 80
NCHUNK = EPW // CK
RPT = 640
f32 = jnp.float32
i32 = jnp.int32


def _front_body(x_ref, w_ref, att_ref, h_ref, ap_ref, mx_ref):
    i = pl.program_id(0)
    h = jnp.dot(x_ref[...], w_ref[...], preferred_element_type=f32)
    h_ref[...] = h
    ap = jnp.dot(h, att_ref[...], preferred_element_type=f32)
    ap_ref[...] = ap
    cur = jnp.max(ap, axis=0, keepdims=True)

    @pl.when(i == 0)
    def _():
        mx_ref[...] = cur

    @pl.when(i != 0)
    def _():
        mx_ref[...] = jnp.maximum(mx_ref[...], cur)


def _front(x, W, att2):
    BN = 1000
    return pl.pallas_call(
        _front_body,
        grid=(N // BN,),
        in_specs=[pl.BlockSpec((BN, D), lambda i: (i, 0)),
                  pl.BlockSpec((D, D), lambda i: (0, 0)),
                  pl.BlockSpec((D, 16), lambda i: (0, 0))],
        out_specs=[pl.BlockSpec((BN, D), lambda i: (i, 0)),
                   pl.BlockSpec((BN, 16), lambda i: (i, 0)),
                   pl.BlockSpec((1, 16), lambda i: (0, 0))],
        out_shape=[jax.ShapeDtypeStruct((N, D), f32),
                   jax.ShapeDtypeStruct((N, 16), f32),
                   jax.ShapeDtypeStruct((1, 16), f32)],
    )(x, W, att2)


_sc_mesh = plsc.VectorSubcoreMesh(core_axis_name="c", subcore_axis_name="s",
                                  num_cores=NC, num_subcores=NS)


@functools.partial(
    pl.kernel,
    out_type=[jax.ShapeDtypeStruct((NC, N, D), f32),
              jax.ShapeDtypeStruct((NC, N, 16), f32)],
    mesh=_sc_mesh,
    compiler_params=pltpu.CompilerParams(needs_layout_passes=False,
                                         use_tc_tiling_on_sc=False),
    scratch_types=[
        pltpu.VMEM((16,), f32),
        pltpu.VMEM((1, 2 * CK), i32),
        pltpu.VMEM((1, 2 * CK), i32),
        pltpu.VMEM((2 * CK, 16), f32),
        pltpu.VMEM((2 * CK, 16), f32),
        pltpu.VMEM((CK, D), f32),
        pltpu.VMEM((CK, D), f32),
        pltpu.VMEM((CK, 16), f32),
        pltpu.VMEM((CK, 16), f32),
        pltpu.VMEM((1, CK), i32),
        pltpu.VMEM((1, CK), i32),
        pltpu.SemaphoreType.DMA,
        pltpu.SemaphoreType.DMA,
        pltpu.SemaphoreType.DMA,
        pltpu.SemaphoreType.DMA,
        pltpu.SemaphoreType.DMA,
        pltpu.SemaphoreType.DMA,
        pltpu.VMEM_SHARED((N, D), f32),
        pltpu.VMEM_SHARED((N, 16), f32),
    ],
)
def _sc_edge(ap_h, c_h, ei_h, h_h,
             out_h, den_h,
             c_t, sd0, sd1, asg0, asg1, rows0, rows1,
             wrep0, wrep1, scx0, scx1,
             isem0, isem1, gsem0, gsem1, ssem0, ssem1,
             out_acc, den_acc):
    cid = lax.axis_index("c")
    sid = lax.axis_index("s")
    wid = cid * NS + sid
    zero16 = jnp.zeros((L,), f32)

    for k in range(CK):
        wrep0[k] = zero16
        for q in range(D // L):
            rows0[k, pl.ds(q * L, L)] = zero16
    nbase = sid * RPT
    for t in range(RPT // CK):
        rb = nbase + t * CK

        @pl.when(rb < N)
        def _():
            pltpu.sync_copy(rows0, out_acc.at[pl.ds(rb, CK)])
            pltpu.sync_copy(wrep0, den_acc.at[pl.ds(rb, CK)])

    pltpu.sync_copy(c_h, c_t)
    c_v = c_t[...]
    iot = lax.iota(i32, L)
    zeros_i = jnp.zeros((L,), i32)
    ones_i = jnp.ones((L,), i32)

    plsc.subcore_barrier()

    gdims = lax.GatherDimensionNumbers(
        offset_dims=(), collapsed_slice_dims=(0,), start_index_map=(0,))

    def _splat(v, k):
        idx = jnp.full((L, 1), k, i32)
        return lax.gather(v, idx, gdims, (1,),
                          mode=lax.GatherScatterMode.PROMISE_IN_BOUNDS)

    bufs = ((sd0, asg0, None, rows0, wrep0, scx0, isem0, gsem0, ssem0),
            (sd1, asg1, None, rows1, wrep1, scx1, isem1, gsem1, ssem1))

    def issue_gathers(b):
        sd, asg, rows = b[0], b[1], b[3]
        gsem = b[7]
        pltpu.async_copy(h_h.at[sd.at[0, pl.ds(0, CK)]], rows, gsem)
        pltpu.async_copy(ap_h.at[sd.at[0]], asg, gsem)

    def wait_gathers(b):
        sd, asg, rows = b[0], b[1], b[3]
        gsem = b[7]
        pltpu.make_async_copy(h_h.at[sd.at[0, pl.ds(0, CK)]], rows, gsem).wait()
        pltpu.make_async_copy(ap_h.at[sd.at[0]], asg, gsem).wait()

    def issue_scatters(b):
        rows, wrep, scx, ssem = b[3], b[4], b[5], b[8]
        pltpu.async_copy(rows, out_acc.at[scx.at[0]], ssem, add=True)
        pltpu.async_copy(wrep, den_acc.at[scx.at[0]], ssem, add=True)

    def wait_scatters(b):
        rows, wrep, scx, ssem = b[3], b[4], b[5], b[8]
        pltpu.make_async_copy(rows, out_acc.at[scx.at[0]], ssem).wait()
        pltpu.make_async_copy(wrep, den_acc.at[scx.at[0]], ssem).wait()

    def process(b):
        sd, asg, rows, wrep, scx = b[0], b[1], b[3], b[4], b[5]
        wvs = []
        for g in range(CK // L):
            kvec = g * L + iot
            scx[0, pl.ds(g * L, L)] = sd[0, pl.ds(CK + g * L, L)]
            e = (plsc.load_gather(asg, [kvec, zeros_i])
                 + plsc.load_gather(asg, [CK + kvec, ones_i]))
            e = jnp.where(e >= 0.0, e, 0.2 * e) - c_v
            wvs.append(jnp.exp(e))
        for g in range(CK // L):
            for k in range(L):
                r = g * L + k
                wk = _splat(wvs[g], k)
                wrep[r] = wk
                for q in range(D // L):
                    rows[r, pl.ds(q * L, L)] = rows[r, pl.ds(q * L, L)] * wk
        issue_scatters(b)

    def step(j, X, Y, wait_scatter):
        if wait_scatter:
            wait_scatters(Y)
        pltpu.make_async_copy(ei_h.at[wid, j + 1], Y[0], Y[6]).wait()
        issue_gathers(Y)
        wait_gathers(X)
        process(X)
        jp2 = jnp.minimum(j + 2, NCHUNK - 1)
        pltpu.async_copy(ei_h.at[wid, jp2], X[0], X[6])

    pltpu.sync_copy(ei_h.at[wid, 0], sd0)
    issue_gathers(bufs[0])
    pltpu.async_copy(ei_h.at[wid, 1], sd1, isem1)
    step(0, bufs[0], bufs[1], wait_scatter=False)
    step(1, bufs[1], bufs[0], wait_scatter=True)

    def pair(p, carry):
        j = 2 * p
        step(j, bufs[0], bufs[1], wait_scatter=True)
        step(j + 1, bufs[1], bufs[0], wait_scatter=True)
        return carry

    lax.fori_loop(1, (NCHUNK - 1) // 2, pair, 0)

    wait_scatters(bufs[1])
    wait_gathers(bufs[0])
    process(bufs[0])
    wait_scatters(bufs[0])
    pltpu.make_async_copy(ei_h.at[wid, NCHUNK - 1], sd1, isem1).wait()

    plsc.subcore_barrier()

    for t in range(RPT // CK):
        rb = nbase + t * CK

        @pl.when(rb < N)
        def _():
            pltpu.sync_copy(out_acc.at[pl.ds(rb, CK)], rows0)
            pltpu.sync_copy(rows0, out_h.at[cid, pl.ds(rb, CK)])
            pltpu.sync_copy(den_acc.at[pl.ds(rb, CK)], wrep0)
            pltpu.sync_copy(wrep0, den_h.at[cid, pl.ds(rb, CK)])


def _mid_body(p_ref, d_ref, b_ref, w_ref, att_ref, h_ref, ap_ref, mx_ref):
    i = pl.program_id(0)
    s = p_ref[0] + p_ref[1]
    den = d_ref[0, :, 0:1] + d_ref[1, :, 0:1]
    y = s / (den + 1e-16) + b_ref[...]
    y = jnp.where(y > 0.0, y, jnp.exp(jnp.minimum(y, 0.0)) - 1.0)
    h = jnp.dot(y, w_ref[...], preferred_element_type=f32)
    h_ref[...] = h
    ap = jnp.dot(h, att_ref[...], preferred_element_type=f32)
    ap_ref[...] = ap
    cur = jnp.max(ap, axis=0, keepdims=True)

    @pl.when(i == 0)
    def _():
        mx_ref[...] = cur

    @pl.when(i != 0)
    def _():
        mx_ref[...] = jnp.maximum(mx_ref[...], cur)


def _mid(p, d, b, W, att2):
    BN = 1000
    return pl.pallas_call(
        _mid_body,
        grid=(N // BN,),
        in_specs=[pl.BlockSpec((2, BN, D), lambda i: (0, i, 0)),
                  pl.BlockSpec((2, BN, 16), lambda i: (0, i, 0)),
                  pl.BlockSpec((1, D), lambda i: (0, 0)),
                  pl.BlockSpec((D, D), lambda i: (0, 0)),
                  pl.BlockSpec((D, 16), lambda i: (0, 0))],
        out_specs=[pl.BlockSpec((BN, D), lambda i: (i, 0)),
                   pl.BlockSpec((BN, 16), lambda i: (i, 0)),
                   pl.BlockSpec((1, 16), lambda i: (0, 0))],
        out_shape=[jax.ShapeDtypeStruct((N, D), f32),
                   jax.ShapeDtypeStruct((N, 16), f32),
                   jax.ShapeDtypeStruct((1, 16), f32)],
    )(p, d, b, W, att2)


def _combine_body(p_ref, d_ref, b_ref, o_ref, *, elu):
    s = p_ref[0] + p_ref[1]
    den = d_ref[0, :, 0:1] + d_ref[1, :, 0:1]
    y = s / (den + 1e-16) + b_ref[...]
    if elu:
        y = jnp.where(y > 0.0, y, jnp.exp(jnp.minimum(y, 0.0)) - 1.0)
    o_ref[...] = y


def _combine(p, d, b, elu):
    BN = 1000
    return pl.pallas_call(
        functools.partial(_combine_body, elu=elu),
        grid=(N // BN,),
        in_specs=[pl.BlockSpec((2, BN, D), lambda i: (0, i, 0)),
                  pl.BlockSpec((2, BN, 16), lambda i: (0, i, 0)),
                  pl.BlockSpec((1, D), lambda i: (0, 0))],
        out_specs=pl.BlockSpec((BN, D), lambda i: (i, 0)),
        out_shape=jax.ShapeDtypeStruct((N, D), f32),
    )(p, d, b)


def _shift(mx):
    c = mx[0, 0] + mx[0, 1]
    c = jnp.where(c >= 0.0, c, 0.2 * c)
    return jnp.broadcast_to(c, (16,))


def kernel(x, edge_index, W1, att_src1, att_dst1, b1, W2, att_src2, att_dst2, b2):
    src = edge_index[0].astype(i32).reshape(NW, NCHUNK, 1, CK)
    dst = edge_index[1].astype(i32).reshape(NW, NCHUNK, 1, CK)
    ei = jnp.concatenate([src, dst], axis=3)
    att21 = jnp.zeros((D, 16), f32).at[:, 0].set(att_src1).at[:, 1].set(att_dst1)
    att22 = jnp.zeros((D, 16), f32).at[:, 0].set(att_src2).at[:, 1].set(att_dst2)

    h1, ap1, mx1 = _front(x, W1, att21)
    outp1, denp1 = _sc_edge(ap1, _shift(mx1), ei, h1)
    h2, ap2, mx2 = _mid(outp1, denp1, b1.reshape(1, D), W2, att22)
    outp2, denp2 = _sc_edge(ap2, _shift(mx2), ei, h2)
    return _combine(outp2, denp2, b2.reshape(1, D), elu=False)
